# baseline (speedup 1.0000x reference)
import functools

import jax
import jax.numpy as jnp
from jax import lax
from jax.experimental import pallas as pl
from jax.experimental.pallas import tpu as pltpu
from jax.experimental.pallas import tpu_sc as plsc

N = 10000
D_IN = 128
HID = 64
OUT = 64
E = 320000
NS = 1000

NP = 10112
ROW = 128
CH = 128
NWORK = 32
CPW = 80
EPW = CPW * CH
EPAD = EPW * NWORK
NSP = 1024
SPW = NSP // NWORK


def _proj_body(x_ref, w_ref, ai_ref, aj_ref, b_ref, tab_ref):
    h = jnp.dot(x_ref[...], w_ref[...], preferred_element_type=jnp.float32)
    ai = jnp.dot(h, ai_ref[...], preferred_element_type=jnp.float32)
    aj = jnp.dot(h, aj_ref[...], preferred_element_type=jnp.float32) + b_ref[0, 0]
    z8 = jnp.zeros_like(ai)
    z40 = jnp.zeros((h.shape[0], 40), jnp.float32)
    tab_ref[...] = jnp.concatenate([h, ai, z8, aj, z40], axis=1)


def _proj(x, W, Ai, Aj, b):
    blk = 2528
    din = x.shape[1]
    return pl.pallas_call(
        _proj_body,
        grid=(NP // blk,),
        in_specs=[
            pl.BlockSpec((blk, din), lambda i: (i, 0)),
            pl.BlockSpec((din, HID), lambda i: (0, 0)),
            pl.BlockSpec((HID, 8), lambda i: (0, 0)),
            pl.BlockSpec((HID, 8), lambda i: (0, 0)),
            pl.BlockSpec(memory_space=pltpu.SMEM),
        ],
        out_specs=pl.BlockSpec((blk, ROW), lambda i: (i, 0)),
        out_shape=jax.ShapeDtypeStruct((NP, ROW), jnp.float32),
    )(x, W, Ai, Aj, b)


def _fin_body(h_heads, out_w, p0_ref, p1_ref, tab_ref, rep_ref, o_ref):
    h = tab_ref[:, 0:64]
    t = tab_ref[:, 64:64 + h_heads] + tab_ref[:, 80:80 + h_heads]
    ws = jnp.exp(jnp.maximum(t, 0.2 * t))
    num = p0_ref[:, 0:64] + p1_ref[:, 0:64]
    den = p0_ref[:, 64:64 + h_heads] + p1_ref[:, 64:64 + h_heads] + ws
    rep = rep_ref[...]
    wsr = jnp.dot(ws, rep, preferred_element_type=jnp.float32)
    denr = jnp.dot(den, rep, preferred_element_type=jnp.float32)
    out = (num + h * wsr) / denr
    if out_w > 64:
        out = jnp.concatenate(
            [out, jnp.zeros((out.shape[0], out_w - 64), jnp.float32)], axis=1)
    o_ref[...] = out


def _fin_relu_body(h_heads, out_w, p0_ref, p1_ref, tab_ref, rep_ref, o_ref):
    _fin_body(h_heads, out_w, p0_ref, p1_ref, tab_ref, rep_ref, o_ref)
    o_ref[...] = jnp.maximum(o_ref[...], 0.0)


def _finalize(p0, p1, tab, rep, h_heads, relu, out_w):
    blk = 2528
    body = _fin_relu_body if relu else _fin_body
    return pl.pallas_call(
        functools.partial(body, h_heads, out_w),
        grid=(NP // blk,),
        in_specs=[
            pl.BlockSpec((blk, ROW), lambda i: (i, 0)),
            pl.BlockSpec((blk, ROW), lambda i: (i, 0)),
            pl.BlockSpec((blk, ROW), lambda i: (i, 0)),
            pl.BlockSpec((h_heads, 64), lambda i: (0, 0)),
        ],
        out_specs=pl.BlockSpec((blk, out_w), lambda i: (i, 0)),
        out_shape=jax.ShapeDtypeStruct((NP, out_w), jnp.float32),
    )(p0, p1, tab, rep)


def _loss_body(sup_ref, lab_ref, o_ref):
    sup = sup_ref[:, 0:OUT]
    m = jnp.max(sup, axis=1, keepdims=True)
    lse = jnp.log(jnp.sum(jnp.exp(sup - m), axis=1, keepdims=True)) + m
    lab = lab_ref[...]
    onehot = lax.broadcasted_iota(jnp.int32, (NSP, OUT), 1) == lab
    picked = jnp.sum(jnp.where(onehot, sup, 0.0), axis=1, keepdims=True)
    logp = picked - lse
    valid = lax.broadcasted_iota(jnp.int32, (NSP, 1), 0) < NS
    o_ref[0, 0] = -jnp.sum(jnp.where(valid, logp, 0.0)) / NS


def _loss(sup, labels2d):
    return pl.pallas_call(
        _loss_body,
        in_specs=[
            pl.BlockSpec((NSP, ROW), lambda: (0, 0)),
            pl.BlockSpec((NSP, 1), lambda: (0, 0)),
        ],
        out_specs=pl.BlockSpec(memory_space=pltpu.SMEM),
        out_shape=jax.ShapeDtypeStruct((1, 1), jnp.float32),
    )(sup, labels2d)


def _gather1d(vec, idx):
    return lax.gather(
        vec, idx[:, None],
        lax.GatherDimensionNumbers(offset_dims=(), collapsed_slice_dims=(0,),
                                   start_index_map=(0,)),
        (1,), mode=lax.GatherScatterMode.PROMISE_IN_BOUNDS)


def _edge_kernel(h_heads, tab, src, dst):
    mesh = plsc.VectorSubcoreMesh(core_axis_name="c", subcore_axis_name="s")

    @functools.partial(
        pl.kernel, mesh=mesh,
        out_type=jax.ShapeDtypeStruct((2, NP, ROW), jnp.float32),
        scratch_types=[
            pltpu.VMEM((CH,), jnp.int32),
            pltpu.VMEM((CH,), jnp.int32),
            pltpu.VMEM((CH, ROW), jnp.float32),
            pltpu.VMEM((CH, ROW), jnp.float32),
            pltpu.VMEM((CH, ROW), jnp.float32),
            pltpu.VMEM_SHARED((NP, ROW), jnp.float32),
            pltpu.SemaphoreType.DMA,
            pltpu.SemaphoreType.DMA,
        ],
    )
    def k(tab_hbm, src_hbm, dst_hbm, out_hbm,
          sidx, didx, rows, drows, msg, acc, sem_g, sem_a):
        c = lax.axis_index("c")
        s = lax.axis_index("s")
        wid = c * 16 + s
        base = wid * EPW

        def zrow(i, carry):
            for kk in range(ROW // 16):
                msg[i, pl.ds(16 * kk, 16)] = jnp.zeros((16,), jnp.float32)
            return carry
        lax.fori_loop(0, CH, zrow, 0)
        rpt = NP // 16
        for i in range(rpt // CH):
            pltpu.sync_copy(msg, acc.at[pl.ds(s * rpt + i * CH, CH)])
        tail = rpt % CH
        if tail:
            pltpu.sync_copy(msg.at[pl.ds(0, tail)],
                            acc.at[pl.ds(s * rpt + (rpt // CH) * CH, tail)])
        plsc.subcore_barrier()

        lane = lax.iota(jnp.int32, 16)
        if h_heads == 8:
            pats = [(lane + 16 * kk) >> 3 for kk in range(4)]
        else:
            pats = [lane * 0 for _ in range(4)]

        def chunk(j, carry):
            off = base + j * CH
            pltpu.sync_copy(src_hbm.at[pl.ds(off, CH)], sidx)
            pltpu.sync_copy(dst_hbm.at[pl.ds(off, CH)], didx)
            cg = pltpu.async_copy(tab_hbm.at[sidx], rows, sem_g)
            ca = pltpu.async_copy(tab_hbm.at[didx], drows, sem_a)
            cg.wait()
            ca.wait()

            def edge(e, cc):
                ai = rows[e, pl.ds(64, 16)]
                aj = drows[e, pl.ds(80, 16)]
                t = ai + aj
                w = jnp.exp(jnp.maximum(t, 0.2 * t))
                for kk in range(4):
                    wb = _gather1d(w, pats[kk])
                    msg[e, pl.ds(16 * kk, 16)] = rows[e, pl.ds(16 * kk, 16)] * wb
                msg[e, pl.ds(64, 16)] = w
                return cc
            lax.fori_loop(0, CH, edge, 0)

            pltpu.sync_copy(msg, acc.at[didx], add=True)
            return carry
        lax.fori_loop(0, CPW, chunk, 0)
        plsc.subcore_barrier()

        for i in range(rpt // CH):
            r0 = s * rpt + i * CH
            pltpu.sync_copy(acc.at[pl.ds(r0, CH)], out_hbm.at[c, pl.ds(r0, CH)])
        if tail:
            r0 = s * rpt + (rpt // CH) * CH
            pltpu.sync_copy(acc.at[pl.ds(r0, tail)], out_hbm.at[c, pl.ds(r0, tail)])

    return k(tab, src, dst)


def _sup_gather(logits_p, sup_idx):
    mesh = plsc.VectorSubcoreMesh(core_axis_name="c", subcore_axis_name="s")

    @functools.partial(
        pl.kernel, mesh=mesh,
        out_type=jax.ShapeDtypeStruct((NSP, ROW), jnp.float32),
        scratch_types=[
            pltpu.VMEM((SPW,), jnp.int32),
            pltpu.VMEM((SPW, ROW), jnp.float32),
            pltpu.SemaphoreType.DMA,
        ],
    )
    def k(tab_hbm, idx_hbm, out_hbm, idxv, rowsv, sem):
        wid = lax.axis_index("c") * 16 + lax.axis_index("s")
        base = wid * SPW
        pltpu.sync_copy(idx_hbm.at[pl.ds(base, SPW)], idxv)
        pltpu.async_copy(tab_hbm.at[idxv], rowsv, sem).wait()
        pltpu.sync_copy(rowsv, out_hbm.at[pl.ds(base, SPW)])

    return k(logits_p, sup_idx)


def _head_mats(aw, hid, heads):
    dh = hid // heads
    Ai = jnp.zeros((hid, 8), jnp.float32)
    Aj = jnp.zeros((hid, 8), jnp.float32)
    for hh in range(heads):
        Ai = Ai.at[hh * dh:(hh + 1) * dh, hh].set(aw[:dh, 0])
        Aj = Aj.at[hh * dh:(hh + 1) * dh, hh].set(aw[dh:2 * dh, 0])
    return Ai, Aj


def _rep_mat(heads):
    rep = jnp.zeros((heads, 64), jnp.float32)
    dh = 64 // heads
    for hh in range(heads):
        rep = rep.at[hh, hh * dh:(hh + 1) * dh].set(1.0)
    return rep


def kernel(x, edge_index, supervised_nodes, labels, W1, a1_w, a1_b, W2, a2_w, a2_b):
    xp = jnp.zeros((NP, D_IN), jnp.float32).at[:N].set(x)
    padE = jnp.full((EPAD - E,), N, jnp.int32)
    srcp = jnp.concatenate([edge_index[0], padE])
    dstp = jnp.concatenate([edge_index[1], padE])
    supp = jnp.concatenate([supervised_nodes, jnp.zeros((NSP - NS,), jnp.int32)])
    lab2d = jnp.zeros((NSP, 1), jnp.int32).at[:NS, 0].set(labels)

    Ai1, Aj1 = _head_mats(a1_w, HID, 8)
    Ai2, Aj2 = _head_mats(a2_w, HID, 1)
    rep1 = _rep_mat(8)
    rep2 = _rep_mat(1)
    b1 = a1_b.reshape(1, 1)
    b2 = a2_b.reshape(1, 1)

    tab1 = _proj(xp, W1, Ai1, Aj1, b1)
    p1 = _edge_kernel(8, tab1, srcp, dstp)
    h2 = _finalize(p1[0], p1[1], tab1, rep1, 8, relu=True, out_w=64)

    tab2 = _proj(h2, W2, Ai2, Aj2, b2)
    p2 = _edge_kernel(1, tab2, srcp, dstp)
    logits_p = _finalize(p2[0], p2[1], tab2, rep2, 1, relu=False, out_w=ROW)

    sup = _sup_gather(logits_p, supp)
    loss = _loss(sup, lab2d)[0, 0]
    return logits_p[:N, :OUT], loss


# speedup vs baseline: 1.3979x; 1.3979x over previous
import functools

import jax
import jax.numpy as jnp
from jax import lax
from jax.experimental import pallas as pl
from jax.experimental.pallas import tpu as pltpu
from jax.experimental.pallas import tpu_sc as plsc

N = 10000
D_IN = 128
HID = 64
OUT = 64
E = 320000
NS = 1000

NP = 10112
ROW = 128
CH = 128
NWORK = 32
CPW = 79
EPW = CPW * CH
EPAD = EPW * NWORK
NSP = 1024
SPW = NSP // NWORK


def _proj_body(x_ref, w_ref, ai_ref, aj_ref, b_ref, tab_ref):
    h = jnp.dot(x_ref[...], w_ref[...], preferred_element_type=jnp.float32)
    ai = jnp.dot(h, ai_ref[...], preferred_element_type=jnp.float32)
    aj = jnp.dot(h, aj_ref[...], preferred_element_type=jnp.float32) + b_ref[0, 0]
    z8 = jnp.zeros_like(ai)
    z40 = jnp.zeros((h.shape[0], 40), jnp.float32)
    tab_ref[...] = jnp.concatenate([h, ai, z8, aj, z40], axis=1)


def _proj(x, W, Ai, Aj, b):
    blk = 2528
    din = x.shape[1]
    return pl.pallas_call(
        _proj_body,
        grid=(NP // blk,),
        in_specs=[
            pl.BlockSpec((blk, din), lambda i: (i, 0)),
            pl.BlockSpec((din, HID), lambda i: (0, 0)),
            pl.BlockSpec((HID, 8), lambda i: (0, 0)),
            pl.BlockSpec((HID, 8), lambda i: (0, 0)),
            pl.BlockSpec(memory_space=pltpu.SMEM),
        ],
        out_specs=pl.BlockSpec((blk, ROW), lambda i: (i, 0)),
        out_shape=jax.ShapeDtypeStruct((NP, ROW), jnp.float32),
    )(x, W, Ai, Aj, b)


def _fin_body(h_heads, out_w, p0_ref, p1_ref, tab_ref, rep_ref, o_ref):
    h = tab_ref[:, 0:64]
    t = tab_ref[:, 64:64 + h_heads] + tab_ref[:, 80:80 + h_heads]
    ws = jnp.exp(jnp.maximum(t, 0.2 * t))
    num = p0_ref[:, 0:64] + p1_ref[:, 0:64]
    den = p0_ref[:, 64:64 + h_heads] + p1_ref[:, 64:64 + h_heads] + ws
    rep = rep_ref[...]
    wsr = jnp.dot(ws, rep, preferred_element_type=jnp.float32)
    denr = jnp.dot(den, rep, preferred_element_type=jnp.float32)
    out = (num + h * wsr) / denr
    if out_w > 64:
        out = jnp.concatenate(
            [out, jnp.zeros((out.shape[0], out_w - 64), jnp.float32)], axis=1)
    o_ref[...] = out


def _fin_relu_body(h_heads, out_w, p0_ref, p1_ref, tab_ref, rep_ref, o_ref):
    _fin_body(h_heads, out_w, p0_ref, p1_ref, tab_ref, rep_ref, o_ref)
    o_ref[...] = jnp.maximum(o_ref[...], 0.0)


def _finalize(p0, p1, tab, rep, h_heads, relu, out_w):
    blk = 2528
    body = _fin_relu_body if relu else _fin_body
    return pl.pallas_call(
        functools.partial(body, h_heads, out_w),
        grid=(NP // blk,),
        in_specs=[
            pl.BlockSpec((blk, ROW), lambda i: (i, 0)),
            pl.BlockSpec((blk, ROW), lambda i: (i, 0)),
            pl.BlockSpec((blk, ROW), lambda i: (i, 0)),
            pl.BlockSpec((h_heads, 64), lambda i: (0, 0)),
        ],
        out_specs=pl.BlockSpec((blk, out_w), lambda i: (i, 0)),
        out_shape=jax.ShapeDtypeStruct((NP, out_w), jnp.float32),
    )(p0, p1, tab, rep)


def _loss_body(sup_ref, lab_ref, o_ref):
    sup = sup_ref[:, 0:OUT]
    m = jnp.max(sup, axis=1, keepdims=True)
    lse = jnp.log(jnp.sum(jnp.exp(sup - m), axis=1, keepdims=True)) + m
    lab = lab_ref[...]
    onehot = lax.broadcasted_iota(jnp.int32, (NSP, OUT), 1) == lab
    picked = jnp.sum(jnp.where(onehot, sup, 0.0), axis=1, keepdims=True)
    logp = picked - lse
    valid = lax.broadcasted_iota(jnp.int32, (NSP, 1), 0) < NS
    o_ref[0, 0] = -jnp.sum(jnp.where(valid, logp, 0.0)) / NS


def _loss(sup, labels2d):
    return pl.pallas_call(
        _loss_body,
        in_specs=[
            pl.BlockSpec((NSP, ROW), lambda: (0, 0)),
            pl.BlockSpec((NSP, 1), lambda: (0, 0)),
        ],
        out_specs=pl.BlockSpec(memory_space=pltpu.SMEM),
        out_shape=jax.ShapeDtypeStruct((1, 1), jnp.float32),
    )(sup, labels2d)


def _gather1d(vec, idx):
    return lax.gather(
        vec, idx[:, None],
        lax.GatherDimensionNumbers(offset_dims=(), collapsed_slice_dims=(0,),
                                   start_index_map=(0,)),
        (1,), mode=lax.GatherScatterMode.PROMISE_IN_BOUNDS)


def _edge_kernel(h_heads, tab, src, dst):
    mesh = plsc.VectorSubcoreMesh(core_axis_name="c", subcore_axis_name="s")

    @functools.partial(
        pl.kernel, mesh=mesh,
        out_type=jax.ShapeDtypeStruct((2, NP, ROW), jnp.float32),
        scratch_types=[
            pltpu.VMEM((CH,), jnp.int32),
            pltpu.VMEM((CH,), jnp.int32),
            pltpu.VMEM((CH, ROW), jnp.float32),
            pltpu.VMEM((CH, ROW), jnp.float32),
            pltpu.VMEM((CH, ROW), jnp.float32),
            pltpu.VMEM_SHARED((NP, ROW), jnp.float32),
            pltpu.SemaphoreType.DMA,
            pltpu.SemaphoreType.DMA,
        ],
    )
    def k(tab_hbm, src_hbm, dst_hbm, out_hbm,
          sidx, didx, rows, drows, msg, acc, sem_g, sem_a):
        c = lax.axis_index("c")
        s = lax.axis_index("s")
        wid = c * 16 + s
        base = wid * EPW

        def zrow(i, carry):
            for kk in range(ROW // 16):
                msg[i, pl.ds(16 * kk, 16)] = jnp.zeros((16,), jnp.float32)
            return carry
        lax.fori_loop(0, CH, zrow, 0)
        rpt = NP // 16
        for i in range(rpt // CH):
            pltpu.sync_copy(msg, acc.at[pl.ds(s * rpt + i * CH, CH)])
        tail = rpt % CH
        if tail:
            pltpu.sync_copy(msg.at[pl.ds(0, tail)],
                            acc.at[pl.ds(s * rpt + (rpt // CH) * CH, tail)])
        plsc.subcore_barrier()

        lane = lax.iota(jnp.int32, 16)
        if h_heads == 8:
            pats = [(lane + 16 * kk) >> 3 for kk in range(4)]
        else:
            pats = [lane * 0 for _ in range(4)]

        def chunk(j, carry):
            off = base + j * CH
            pltpu.sync_copy(src_hbm.at[pl.ds(off, CH)], sidx)
            pltpu.sync_copy(dst_hbm.at[pl.ds(off, CH)], didx)
            cg = pltpu.async_copy(tab_hbm.at[sidx], rows, sem_g)
            ca = pltpu.async_copy(tab_hbm.at[didx], drows, sem_a)
            cg.wait()
            ca.wait()

            def edge(e, cc):
                ai = rows[e, pl.ds(64, 16)]
                aj = drows[e, pl.ds(80, 16)]
                t = ai + aj
                w = jnp.exp(jnp.maximum(t, 0.2 * t))
                for kk in range(4):
                    wb = _gather1d(w, pats[kk])
                    msg[e, pl.ds(16 * kk, 16)] = rows[e, pl.ds(16 * kk, 16)] * wb
                msg[e, pl.ds(64, 16)] = w
                return cc
            lax.fori_loop(0, CH, edge, 0)

            pltpu.sync_copy(msg, acc.at[didx], add=True)
            return carry
        lax.fori_loop(0, CPW, chunk, 0)
        plsc.subcore_barrier()

        for i in range(rpt // CH):
            r0 = s * rpt + i * CH
            pltpu.sync_copy(acc.at[pl.ds(r0, CH)], out_hbm.at[c, pl.ds(r0, CH)])
        if tail:
            r0 = s * rpt + (rpt // CH) * CH
            pltpu.sync_copy(acc.at[pl.ds(r0, tail)], out_hbm.at[c, pl.ds(r0, tail)])

    return k(tab, src, dst)


def _sup_gather(logits_p, sup_idx):
    mesh = plsc.VectorSubcoreMesh(core_axis_name="c", subcore_axis_name="s")

    @functools.partial(
        pl.kernel, mesh=mesh,
        out_type=jax.ShapeDtypeStruct((NSP, ROW), jnp.float32),
        scratch_types=[
            pltpu.VMEM((SPW,), jnp.int32),
            pltpu.VMEM((SPW, ROW), jnp.float32),
            pltpu.SemaphoreType.DMA,
        ],
    )
    def k(tab_hbm, idx_hbm, out_hbm, idxv, rowsv, sem):
        wid = lax.axis_index("c") * 16 + lax.axis_index("s")
        base = wid * SPW
        pltpu.sync_copy(idx_hbm.at[pl.ds(base, SPW)], idxv)
        pltpu.async_copy(tab_hbm.at[idxv], rowsv, sem).wait()
        pltpu.sync_copy(rowsv, out_hbm.at[pl.ds(base, SPW)])

    return k(logits_p, sup_idx)


def _head_mats(aw, hid, heads):
    dh = hid // heads
    Ai = jnp.zeros((hid, 8), jnp.float32)
    Aj = jnp.zeros((hid, 8), jnp.float32)
    for hh in range(heads):
        Ai = Ai.at[hh * dh:(hh + 1) * dh, hh].set(aw[:dh, 0])
        Aj = Aj.at[hh * dh:(hh + 1) * dh, hh].set(aw[dh:2 * dh, 0])
    return Ai, Aj


def _rep_mat(heads):
    rep = jnp.zeros((heads, 64), jnp.float32)
    dh = 64 // heads
    for hh in range(heads):
        rep = rep.at[hh, hh * dh:(hh + 1) * dh].set(1.0)
    return rep


def kernel(x, edge_index, supervised_nodes, labels, W1, a1_w, a1_b, W2, a2_w, a2_b):
    xp = jnp.zeros((NP, D_IN), jnp.float32).at[:N].set(x)
    padE = jnp.full((EPAD - E,), N, jnp.int32)
    srcp = jnp.concatenate([edge_index[0], padE])
    dstp = jnp.concatenate([edge_index[1], padE])
    supp = jnp.concatenate([supervised_nodes, jnp.zeros((NSP - NS,), jnp.int32)])
    lab2d = jnp.zeros((NSP, 1), jnp.int32).at[:NS, 0].set(labels)

    Ai1, Aj1 = _head_mats(a1_w, HID, 8)
    Ai2, Aj2 = _head_mats(a2_w, HID, 1)
    rep1 = _rep_mat(8)
    rep2 = _rep_mat(1)
    b1 = a1_b.reshape(1, 1)
    b2 = a2_b.reshape(1, 1)

    tab1 = _proj(xp, W1, Ai1, Aj1, b1)
    p1 = _edge_kernel(8, tab1, srcp, dstp)
    h2 = _finalize(p1[0], p1[1], tab1, rep1, 8, relu=True, out_w=64)

    tab2 = _proj(h2, W2, Ai2, Aj2, b2)
    p2 = _edge_kernel(1, tab2, srcp, dstp)
    logits_p = _finalize(p2[0], p2[1], tab2, rep2, 1, relu=False, out_w=ROW)

    sup = _sup_gather(logits_p, supp)
    loss = _loss(sup, lab2d)[0, 0]
    return logits_p[:N, :OUT], loss
